# Initial kernel scaffold; baseline (speedup 1.0000x reference)
import jax
import jax.numpy as jnp
from jax.experimental import pallas as pl


def kernel(x, edge_index, W1, b1, W2, b2):
    raise NotImplementedError("write your pallas kernel here")


# speedup vs baseline: 7.7648x; 7.7648x over previous
import jax
import jax.numpy as jnp
from jax import lax
from jax.experimental import pallas as pl
from jax.experimental.pallas import tpu as pltpu
from jax.experimental.pallas import tpu_sc as plsc

N = 10000
D = 128
E = 320000

NC = 2
NS = 16
NW = NC * NS
CH = 128
NCHUNK = 80
GRP = 16
NGRP = NCHUNK // GRP
EPW = CH * NCHUNK
E_PAD = EPW * NW
N_PAD = 10112
RPT = N_PAD // NS

_MESH = dict(core_axis_name="c", subcore_axis_name="s",
             num_cores=NC, num_subcores=NS)


def _sc_scatter_body(src_hbm, dst_hbm, y_hbm, zero_hbm, out_hbm,
                     src_g, dst_g, rows0, rows1, acc, sem0, sem1):
    cid = lax.axis_index("c")
    sid = lax.axis_index("s")
    wid = cid * NS + sid

    pltpu.sync_copy(zero_hbm, acc.at[pl.ds(sid * RPT, RPT)])
    plsc.subcore_barrier()

    def group(g, carry):
        pltpu.sync_copy(src_hbm.at[wid, pl.ds(g * GRP, GRP)], src_g)
        pltpu.sync_copy(dst_hbm.at[wid, pl.ds(g * GRP, GRP)], dst_g)
        pltpu.async_copy(y_hbm.at[src_g.at[0]], rows0, sem0)

        def pair(p, c2):
            k = p * 2
            pltpu.async_copy(y_hbm.at[src_g.at[k + 1]], rows1, sem1)
            pltpu.make_async_copy(y_hbm.at[pl.ds(0, CH)], rows0, sem0).wait()
            pltpu.sync_copy(rows0, acc.at[dst_g.at[k]], add=True)
            nk = jnp.where(k + 2 < GRP, k + 2, 0)
            pltpu.async_copy(y_hbm.at[src_g.at[nk]], rows0, sem0)
            pltpu.make_async_copy(y_hbm.at[pl.ds(0, CH)], rows1, sem1).wait()
            pltpu.sync_copy(rows1, acc.at[dst_g.at[k + 1]], add=True)
            return c2

        lax.fori_loop(0, GRP // 2, pair, 0)
        pltpu.make_async_copy(y_hbm.at[pl.ds(0, CH)], rows0, sem0).wait()
        return carry

    lax.fori_loop(0, NGRP, group, 0)

    plsc.subcore_barrier()
    pltpu.sync_copy(acc.at[pl.ds(sid * RPT, RPT)],
                    out_hbm.at[cid, pl.ds(sid * RPT, RPT)])


def _sc_scatter(src3, dst3, y, zero_rows):
    return pl.kernel(
        _sc_scatter_body,
        out_type=jax.ShapeDtypeStruct((NC, N_PAD, D), jnp.float32),
        mesh=plsc.VectorSubcoreMesh(**_MESH),
        scratch_types=[
            pltpu.VMEM((GRP, CH), jnp.int32),
            pltpu.VMEM((GRP, CH), jnp.int32),
            pltpu.VMEM((CH, D), jnp.float32),
            pltpu.VMEM((CH, D), jnp.float32),
            pltpu.VMEM_SHARED((N_PAD, D), jnp.float32),
            pltpu.SemaphoreType.DMA,
            pltpu.SemaphoreType.DMA,
        ],
    )(src3, dst3, y, zero_rows)


def _sc_deg_body(dst_hbm, zero_hbm, ones_hbm, out_hbm, dst_all, ones_v, acc):
    cid = lax.axis_index("c")
    sid = lax.axis_index("s")
    wid = cid * NS + sid

    pltpu.sync_copy(zero_hbm, acc.at[pl.ds(sid * RPT, RPT)])
    pltpu.sync_copy(dst_hbm.at[wid], dst_all)
    pltpu.sync_copy(ones_hbm, ones_v)
    plsc.subcore_barrier()

    def chunk(ch, carry):
        pltpu.sync_copy(ones_v, acc.at[dst_all.at[ch]], add=True)
        return carry

    lax.fori_loop(0, NCHUNK, chunk, 0)

    plsc.subcore_barrier()
    pltpu.sync_copy(acc.at[pl.ds(sid * RPT, RPT)],
                    out_hbm.at[cid, pl.ds(sid * RPT, RPT)])


def _sc_deg(dst3, zero16, ones16):
    return pl.kernel(
        _sc_deg_body,
        out_type=jax.ShapeDtypeStruct((NC, N_PAD, D), jnp.float32),
        mesh=plsc.VectorSubcoreMesh(**_MESH),
        scratch_types=[
            pltpu.VMEM((NCHUNK, CH), jnp.int32),
            pltpu.VMEM((CH, D), jnp.float32),
            pltpu.VMEM_SHARED((N_PAD, D), jnp.float32),
        ],
    )(dst3, zero16, ones16)


R = 2000
G = N // R


def _tc_a1_body(x_ref, w_ref, degp_ref, y_ref, dis_ref):
    deg = degp_ref[0, :, 0:1] + degp_ref[1, :, 0:1] + 1.0
    dis = lax.rsqrt(deg)
    xw = jnp.dot(x_ref[...], w_ref[...], preferred_element_type=jnp.float32)
    y_ref[...] = dis * xw
    dis_ref[...] = jnp.broadcast_to(dis, xw.shape)


def _tc_a1(x, W1, degp):
    return pl.pallas_call(
        _tc_a1_body,
        grid=(G,),
        in_specs=[
            pl.BlockSpec((R, D), lambda i: (i, 0)),
            pl.BlockSpec((D, D), lambda i: (0, 0)),
            pl.BlockSpec((NC, R, D), lambda i: (0, i, 0)),
        ],
        out_specs=[
            pl.BlockSpec((R, D), lambda i: (i, 0)),
            pl.BlockSpec((R, D), lambda i: (i, 0)),
        ],
        out_shape=[
            jax.ShapeDtypeStruct((N, D), jnp.float32),
            jax.ShapeDtypeStruct((N, D), jnp.float32),
        ],
    )(x, W1, degp)


def _tc_a2_body(aggp_ref, y_ref, dis_ref, b_ref, w_ref, out_ref):
    s = aggp_ref[0] + aggp_ref[1] + y_ref[...]
    h = jnp.maximum(dis_ref[...] * s + b_ref[...], 0.0)
    out_ref[...] = dis_ref[...] * jnp.dot(
        h, w_ref[...], preferred_element_type=jnp.float32)


def _tc_a2(aggp, y1, dis, b1, W2):
    return pl.pallas_call(
        _tc_a2_body,
        grid=(G,),
        in_specs=[
            pl.BlockSpec((NC, R, D), lambda i: (0, i, 0)),
            pl.BlockSpec((R, D), lambda i: (i, 0)),
            pl.BlockSpec((R, D), lambda i: (i, 0)),
            pl.BlockSpec((1, D), lambda i: (0, 0)),
            pl.BlockSpec((D, D), lambda i: (0, 0)),
        ],
        out_specs=pl.BlockSpec((R, D), lambda i: (i, 0)),
        out_shape=jax.ShapeDtypeStruct((N, D), jnp.float32),
    )(aggp, y1, dis, b1, W2)


def _tc_a3_body(aggp_ref, y_ref, dis_ref, b_ref, out_ref):
    z = dis_ref[...] * (aggp_ref[0] + aggp_ref[1] + y_ref[...]) + b_ref[...]
    m = jnp.max(z, axis=1, keepdims=True)
    s = jnp.sum(jnp.exp(z - m), axis=1, keepdims=True)
    out_ref[...] = z - m - jnp.log(s)


def _tc_a3(aggp, y2, dis, b2):
    return pl.pallas_call(
        _tc_a3_body,
        grid=(G,),
        in_specs=[
            pl.BlockSpec((NC, R, D), lambda i: (0, i, 0)),
            pl.BlockSpec((R, D), lambda i: (i, 0)),
            pl.BlockSpec((R, D), lambda i: (i, 0)),
            pl.BlockSpec((1, D), lambda i: (0, 0)),
        ],
        out_specs=pl.BlockSpec((R, D), lambda i: (i, 0)),
        out_shape=jax.ShapeDtypeStruct((N, D), jnp.float32),
    )(aggp, y2, dis, b2)


def kernel(x, edge_index, W1, b1, W2, b2):
    src = edge_index[0]
    dst = edge_index[1]
    pad = E_PAD - E
    src_p = jnp.concatenate([src, jnp.zeros((pad,), jnp.int32)])
    dst_p = jnp.concatenate([dst, jnp.full((pad,), N, jnp.int32)])
    src3 = src_p.reshape(NW, NCHUNK, CH)
    dst3 = dst_p.reshape(NW, NCHUNK, CH)

    zero_rows = jnp.zeros((RPT, D), jnp.float32)
    zero16 = jnp.zeros((RPT, D), jnp.float32)
    ones16 = jnp.ones((CH, D), jnp.float32)

    degp = _sc_deg(dst3, zero16, ones16)
    y1, dis = _tc_a1(x, W1, degp[:, :N, :])
    agg1 = _sc_scatter(src3, dst3, y1, zero_rows)
    y2 = _tc_a2(agg1[:, :N, :], y1, dis, b1.reshape(1, D), W2)
    agg2 = _sc_scatter(src3, dst3, y2, zero_rows)
    return _tc_a3(agg2[:, :N, :], y2, dis, b2.reshape(1, D))


# speedup vs baseline: 26.3818x; 3.3976x over previous
import jax
import jax.numpy as jnp
from jax import lax
from jax.experimental import pallas as pl
from jax.experimental.pallas import tpu as pltpu
from jax.experimental.pallas import tpu_sc as plsc

N = 10000
D = 128
E = 320000

NC = 2
NS = 16
NW = NC * NS
CH = 128
NCHUNK = 80
EPW = CH * NCHUNK
CHS = 64
NB = 4
NCHS = EPW // CHS
GRPS = 32
NGRPS = NCHS // GRPS
E_PAD = EPW * NW
N_PAD = 10112
RPT = N_PAD // NS

_MESH = dict(core_axis_name="c", subcore_axis_name="s",
             num_cores=NC, num_subcores=NS)


def _sc_scatter_body(src_hbm, dst_hbm, y_hbm, zero_hbm, out_hbm,
                     src_g, dst_g, r0, r1, r2, r3, acc, s0, s1, s2, s3):
    rows = (r0, r1, r2, r3)
    sems = (s0, s1, s2, s3)
    cid = lax.axis_index("c")
    sid = lax.axis_index("s")
    wid = cid * NS + sid

    pltpu.sync_copy(zero_hbm, acc.at[pl.ds(sid * RPT, RPT)])
    plsc.subcore_barrier()

    def group(g, carry):
        pltpu.sync_copy(src_hbm.at[wid, pl.ds(g * GRPS, GRPS)], src_g)
        pltpu.sync_copy(dst_hbm.at[wid, pl.ds(g * GRPS, GRPS)], dst_g)
        for b in range(NB):
            pltpu.async_copy(y_hbm.at[src_g.at[b]], rows[b], sems[b])

        def quad(q, c2):
            k = q * NB
            for b in range(NB):
                pltpu.make_async_copy(
                    y_hbm.at[pl.ds(0, CHS)], rows[b], sems[b]).wait()
                pltpu.sync_copy(rows[b], acc.at[dst_g.at[k + b]], add=True)
                pltpu.async_copy(
                    y_hbm.at[src_g.at[k + NB + b]], rows[b], sems[b])
            return c2

        lax.fori_loop(0, (GRPS - NB) // NB, quad, 0)
        for b in range(NB):
            pltpu.make_async_copy(
                y_hbm.at[pl.ds(0, CHS)], rows[b], sems[b]).wait()
            pltpu.sync_copy(rows[b], acc.at[dst_g.at[GRPS - NB + b]], add=True)
        return carry

    lax.fori_loop(0, NGRPS, group, 0)

    plsc.subcore_barrier()
    pltpu.sync_copy(acc.at[pl.ds(sid * RPT, RPT)],
                    out_hbm.at[cid, pl.ds(sid * RPT, RPT)])


def _sc_scatter(src3, dst3, y, zero_rows):
    return pl.kernel(
        _sc_scatter_body,
        out_type=jax.ShapeDtypeStruct((NC, N_PAD, D), jnp.float32),
        mesh=plsc.VectorSubcoreMesh(**_MESH),
        scratch_types=[
            pltpu.VMEM((GRPS, CHS), jnp.int32),
            pltpu.VMEM((GRPS, CHS), jnp.int32),
            pltpu.VMEM((CHS, D), jnp.float32),
            pltpu.VMEM((CHS, D), jnp.float32),
            pltpu.VMEM((CHS, D), jnp.float32),
            pltpu.VMEM((CHS, D), jnp.float32),
            pltpu.VMEM_SHARED((N_PAD, D), jnp.float32),
            pltpu.SemaphoreType.DMA,
            pltpu.SemaphoreType.DMA,
            pltpu.SemaphoreType.DMA,
            pltpu.SemaphoreType.DMA,
        ],
    )(src3, dst3, y, zero_rows)


def _sc_deg_body(dst_hbm, zero_hbm, ones_hbm, out_hbm, dst_all, ones_v, acc):
    cid = lax.axis_index("c")
    sid = lax.axis_index("s")
    wid = cid * NS + sid

    pltpu.sync_copy(zero_hbm, acc.at[pl.ds(sid * RPT, RPT)])
    pltpu.sync_copy(dst_hbm.at[wid], dst_all)
    pltpu.sync_copy(ones_hbm, ones_v)
    plsc.subcore_barrier()

    def chunk(ch, carry):
        pltpu.sync_copy(ones_v, acc.at[dst_all.at[ch]], add=True)
        return carry

    lax.fori_loop(0, NCHUNK, chunk, 0)

    plsc.subcore_barrier()
    pltpu.sync_copy(acc.at[pl.ds(sid * RPT, RPT)],
                    out_hbm.at[cid, pl.ds(sid * RPT, RPT)])


def _sc_deg(dst3, zero16, ones16):
    return pl.kernel(
        _sc_deg_body,
        out_type=jax.ShapeDtypeStruct((NC, N_PAD, D), jnp.float32),
        mesh=plsc.VectorSubcoreMesh(**_MESH),
        scratch_types=[
            pltpu.VMEM((NCHUNK, CH), jnp.int32),
            pltpu.VMEM((CH, D), jnp.float32),
            pltpu.VMEM_SHARED((N_PAD, D), jnp.float32),
        ],
    )(dst3, zero16, ones16)


R = 2000
G = N // R


def _tc_a1_body(x_ref, w_ref, degp_ref, y_ref, dis_ref):
    deg = degp_ref[0, :, 0:1] + degp_ref[1, :, 0:1] + 1.0
    dis = lax.rsqrt(deg)
    xw = jnp.dot(x_ref[...], w_ref[...], preferred_element_type=jnp.float32)
    y_ref[...] = dis * xw
    dis_ref[...] = jnp.broadcast_to(dis, xw.shape)


def _tc_a1(x, W1, degp):
    return pl.pallas_call(
        _tc_a1_body,
        grid=(G,),
        in_specs=[
            pl.BlockSpec((R, D), lambda i: (i, 0)),
            pl.BlockSpec((D, D), lambda i: (0, 0)),
            pl.BlockSpec((NC, R, D), lambda i: (0, i, 0)),
        ],
        out_specs=[
            pl.BlockSpec((R, D), lambda i: (i, 0)),
            pl.BlockSpec((R, D), lambda i: (i, 0)),
        ],
        out_shape=[
            jax.ShapeDtypeStruct((N, D), jnp.float32),
            jax.ShapeDtypeStruct((N, D), jnp.float32),
        ],
    )(x, W1, degp)


def _tc_a2_body(aggp_ref, y_ref, dis_ref, b_ref, w_ref, out_ref):
    s = aggp_ref[0] + aggp_ref[1] + y_ref[...]
    h = jnp.maximum(dis_ref[...] * s + b_ref[...], 0.0)
    out_ref[...] = dis_ref[...] * jnp.dot(
        h, w_ref[...], preferred_element_type=jnp.float32)


def _tc_a2(aggp, y1, dis, b1, W2):
    return pl.pallas_call(
        _tc_a2_body,
        grid=(G,),
        in_specs=[
            pl.BlockSpec((NC, R, D), lambda i: (0, i, 0)),
            pl.BlockSpec((R, D), lambda i: (i, 0)),
            pl.BlockSpec((R, D), lambda i: (i, 0)),
            pl.BlockSpec((1, D), lambda i: (0, 0)),
            pl.BlockSpec((D, D), lambda i: (0, 0)),
        ],
        out_specs=pl.BlockSpec((R, D), lambda i: (i, 0)),
        out_shape=jax.ShapeDtypeStruct((N, D), jnp.float32),
    )(aggp, y1, dis, b1, W2)


def _tc_a3_body(aggp_ref, y_ref, dis_ref, b_ref, out_ref):
    z = dis_ref[...] * (aggp_ref[0] + aggp_ref[1] + y_ref[...]) + b_ref[...]
    m = jnp.max(z, axis=1, keepdims=True)
    s = jnp.sum(jnp.exp(z - m), axis=1, keepdims=True)
    out_ref[...] = z - m - jnp.log(s)


def _tc_a3(aggp, y2, dis, b2):
    return pl.pallas_call(
        _tc_a3_body,
        grid=(G,),
        in_specs=[
            pl.BlockSpec((NC, R, D), lambda i: (0, i, 0)),
            pl.BlockSpec((R, D), lambda i: (i, 0)),
            pl.BlockSpec((R, D), lambda i: (i, 0)),
            pl.BlockSpec((1, D), lambda i: (0, 0)),
        ],
        out_specs=pl.BlockSpec((R, D), lambda i: (i, 0)),
        out_shape=jax.ShapeDtypeStruct((N, D), jnp.float32),
    )(aggp, y2, dis, b2)


def kernel(x, edge_index, W1, b1, W2, b2):
    src = edge_index[0]
    dst = edge_index[1]
    pad = E_PAD - E
    ar = jnp.arange(pad, dtype=jnp.int32)
    src_p = jnp.concatenate([src, (ar * 13) % N])
    dst_p = jnp.concatenate([dst, N + (ar * 7) % (N_PAD - N)])
    src3 = src_p.reshape(NW, NCHS, CHS)
    dst3 = dst_p.reshape(NW, NCHS, CHS)
    dst3_deg = dst_p.reshape(NW, NCHUNK, CH)

    zero_rows = jnp.zeros((RPT, D), jnp.float32)
    zero16 = jnp.zeros((RPT, D), jnp.float32)
    ones16 = jnp.ones((CH, D), jnp.float32)

    degp = _sc_deg(dst3_deg, zero16, ones16)
    y1, dis = _tc_a1(x, W1, degp[:, :N, :])
    agg1 = _sc_scatter(src3, dst3, y1, zero_rows)
    y2 = _tc_a2(agg1[:, :N, :], y1, dis, b1.reshape(1, D), W2)
    agg2 = _sc_scatter(src3, dst3, y2, zero_rows)
    return _tc_a3(agg2[:, :N, :], y2, dis, b2.reshape(1, D))
